# baseline (speedup 1.0000x reference)
import functools

import jax
import jax.numpy as jnp
from jax import lax
from jax.experimental import pallas as pl

N = 10000
E = 320000
H = 128

BLK = 1000


def _stage1_body(x_ref, wcat_ref, bcat_ref, wext_ref, out_ref):
    x1 = jnp.maximum(
        jnp.dot(x_ref[...], wcat_ref[...], preferred_element_type=jnp.float32)
        + bcat_ref[...], 0.0)
    out_ref[...] = jnp.dot(x1, wext_ref[...], preferred_element_type=jnp.float32)


def _stage2_body(a0_ref, a1_ref, b_ref, wext_ref, out_ref):
    x = jnp.maximum(a0_ref[...] + a1_ref[...] + b_ref[...], 0.0)
    out_ref[...] = jnp.dot(x, wext_ref[...], preferred_element_type=jnp.float32)


def _stage3_body(a0_ref, a1_ref, b_ref, wp1_ref, bp1_ref, wp2_ref, bp2_ref, out_ref):
    x = jnp.maximum(a0_ref[...] + a1_ref[...] + b_ref[...], 0.0)
    hm = jnp.maximum(
        jnp.dot(x, wp1_ref[...], preferred_element_type=jnp.float32) + bp1_ref[...],
        0.0)
    out_ref[...] = (
        jnp.dot(hm, wp2_ref[...], preferred_element_type=jnp.float32) + bp2_ref[...])


def _tc_stage1(x, wcat, bcat, wext):
    return pl.pallas_call(
        _stage1_body,
        grid=(N // BLK,),
        in_specs=[
            pl.BlockSpec((BLK, H), lambda i: (i, 0)),
            pl.BlockSpec((H, H), lambda i: (0, 0)),
            pl.BlockSpec((1, H), lambda i: (0, 0)),
            pl.BlockSpec((H, 2 * H), lambda i: (0, 0)),
        ],
        out_specs=pl.BlockSpec((BLK, 2 * H), lambda i: (i, 0)),
        out_shape=jax.ShapeDtypeStruct((N, 2 * H), jnp.float32),
    )(x, wcat, bcat, wext)


def _tc_stage2(a0, a1, b, wext):
    return pl.pallas_call(
        _stage2_body,
        grid=(N // BLK,),
        in_specs=[
            pl.BlockSpec((BLK, H), lambda i: (i, 0)),
            pl.BlockSpec((BLK, H), lambda i: (i, 0)),
            pl.BlockSpec((1, H), lambda i: (0, 0)),
            pl.BlockSpec((H, 2 * H), lambda i: (0, 0)),
        ],
        out_specs=pl.BlockSpec((BLK, 2 * H), lambda i: (i, 0)),
        out_shape=jax.ShapeDtypeStruct((N, 2 * H), jnp.float32),
    )(a0, a1, b, wext)


def _tc_stage3(a0, a1, b, wp1, bp1, wp2, bp2):
    return pl.pallas_call(
        _stage3_body,
        grid=(N // BLK,),
        in_specs=[
            pl.BlockSpec((BLK, H), lambda i: (i, 0)),
            pl.BlockSpec((BLK, H), lambda i: (i, 0)),
            pl.BlockSpec((1, H), lambda i: (0, 0)),
            pl.BlockSpec((H, H), lambda i: (0, 0)),
            pl.BlockSpec((1, H), lambda i: (0, 0)),
            pl.BlockSpec((H, 8), lambda i: (0, 0)),
            pl.BlockSpec((1, 8), lambda i: (0, 0)),
        ],
        out_specs=pl.BlockSpec((BLK, 8), lambda i: (i, 0)),
        out_shape=jax.ShapeDtypeStruct((N, 8), jnp.float32),
    )(a0, a1, b, wp1, bp1, wp2, bp2)


def _edge_phase_xla(h, s_src, s_dst, src, dst):
    z = s_src[src] + s_dst[dst]
    a = jnp.where(z > 0, z, 0.2 * z)
    c = jnp.maximum(jnp.max(s_src) + jnp.max(s_dst), 0.0)
    ex = jnp.exp(a - c)
    denom = jax.ops.segment_sum(ex, dst, num_segments=N)
    alpha = ex / jnp.maximum(denom[dst], 1e-16)
    return jax.ops.segment_sum(alpha[:, None] * h[src], dst, num_segments=N)


def kernel(discrete_x, continous_x, edge_index, edge_attr, churn_date, t,
           W_c, b_c, W0, b0, W1, a_src1, a_dst1, b1,
           W2, a_src2, a_dst2, b2, Wp1, bp1, Wp2, bp2):
    W0a = W0[:32]
    W0b = W0[32:].reshape(3, 32, H)
    Wfold = jnp.einsum("ij,gjk->gik", W_c, W0b).reshape(96, H)
    Wcat = jnp.concatenate([W0a, Wfold], axis=0)
    bcat = (b0 + jnp.einsum("j,gjk->k", b_c, W0b))[None]

    def wext(W, a_s, a_d):
        cols = jnp.zeros((H, H), jnp.float32)
        cols = cols.at[:, 0].set(W @ a_s).at[:, 1].set(W @ a_d)
        return jnp.concatenate([W, cols], axis=1)

    Wext1 = wext(W1, a_src1, a_dst1)
    Wext2 = wext(W2, a_src2, a_dst2)
    Wp2p = jnp.zeros((H, 8), jnp.float32).at[:, 0].set(Wp2[:, 0])
    bp2p = jnp.zeros((1, 8), jnp.float32).at[0, 0].set(bp2[0])

    X = jnp.concatenate([discrete_x, continous_x.reshape(N, 96)], axis=1)
    src = edge_index[0]
    dst = edge_index[1]

    o1 = _tc_stage1(X, Wcat, bcat, Wext1)
    h1, s_src1v, s_dst1v = o1[:, :H], o1[:, H], o1[:, H + 1]

    agg1 = _edge_phase_xla(h1, s_src1v, s_dst1v, src, dst)

    o2 = _tc_stage2(agg1, jnp.zeros_like(agg1), b1[None], Wext2)
    h2, s_src2v, s_dst2v = o2[:, :H], o2[:, H], o2[:, H + 1]

    agg2 = _edge_phase_xla(h2, s_src2v, s_dst2v, src, dst)

    o3 = _tc_stage3(agg2, jnp.zeros_like(agg2), b2[None], Wp1, bp1[None],
                    Wp2p, bp2p)
    return o3[:, :1]


# speedup vs baseline: 9.3525x; 9.3525x over previous
import functools

import jax
import jax.numpy as jnp
from jax import lax
from jax.experimental import pallas as pl
from jax.experimental.pallas import tpu as pltpu
from jax.experimental.pallas import tpu_sc as plsc

N = 10000
E = 320000
H = 128

BLK = 1000

CHUNK = 128
EROWS = 2560
EPAD = EROWS * CHUNK
RPS = EROWS // 16
ROWN = RPS // 2
NP = 10240


def _split_out(out, h_ref, s_ref, mx_ref):
    h_ref[...] = out[:, :H]
    s_blk = out[:, H:]
    s_ref[...] = s_blk
    cur = jnp.max(s_blk, axis=0, keepdims=True)

    @pl.when(pl.program_id(0) == 0)
    def _init():
        mx_ref[...] = cur

    @pl.when(pl.program_id(0) != 0)
    def _acc():
        mx_ref[...] = jnp.maximum(mx_ref[...], cur)


def _stage1_body(x_ref, wcat_ref, bcat_ref, wext_ref, h_ref, s_ref, mx_ref):
    x1 = jnp.maximum(
        jnp.dot(x_ref[...], wcat_ref[...], preferred_element_type=jnp.float32)
        + bcat_ref[...], 0.0)
    out = jnp.dot(x1, wext_ref[...], preferred_element_type=jnp.float32)
    _split_out(out, h_ref, s_ref, mx_ref)


def _stage2_body(a0_ref, a1_ref, b_ref, wext_ref, h_ref, s_ref, mx_ref):
    x = jnp.maximum(a0_ref[...] + a1_ref[...] + b_ref[...], 0.0)
    out = jnp.dot(x, wext_ref[...], preferred_element_type=jnp.float32)
    _split_out(out, h_ref, s_ref, mx_ref)


def _stage3_body(a0_ref, a1_ref, b_ref, wp1_ref, bp1_ref, wp2_ref, bp2_ref,
                 out_ref):
    x = jnp.maximum(a0_ref[...] + a1_ref[...] + b_ref[...], 0.0)
    hm = jnp.maximum(
        jnp.dot(x, wp1_ref[...], preferred_element_type=jnp.float32)
        + bp1_ref[...], 0.0)
    out_ref[...] = (
        jnp.dot(hm, wp2_ref[...], preferred_element_type=jnp.float32)
        + bp2_ref[...])


_BLK_SPEC = pl.BlockSpec((BLK, H), lambda i: (i, 0))
_ROW_SPEC = pl.BlockSpec((1, H), lambda i: (0, 0))
_STAGE_OUT_SPECS = [
    _BLK_SPEC,
    _BLK_SPEC,
    pl.BlockSpec((1, H), lambda i: (0, 0)),
]
_STAGE_OUT_SHAPES = [
    jax.ShapeDtypeStruct((N, H), jnp.float32),
    jax.ShapeDtypeStruct((N, H), jnp.float32),
    jax.ShapeDtypeStruct((1, H), jnp.float32),
]


def _tc_stage1(x, wcat, bcat, wext):
    return pl.pallas_call(
        _stage1_body,
        grid=(N // BLK,),
        in_specs=[
            _BLK_SPEC,
            pl.BlockSpec((H, H), lambda i: (0, 0)),
            _ROW_SPEC,
            pl.BlockSpec((H, 2 * H), lambda i: (0, 0)),
        ],
        out_specs=_STAGE_OUT_SPECS,
        out_shape=_STAGE_OUT_SHAPES,
    )(x, wcat, bcat, wext)


def _tc_stage2(a0, a1, b, wext):
    return pl.pallas_call(
        _stage2_body,
        grid=(N // BLK,),
        in_specs=[_BLK_SPEC, _BLK_SPEC, _ROW_SPEC,
                  pl.BlockSpec((H, 2 * H), lambda i: (0, 0))],
        out_specs=_STAGE_OUT_SPECS,
        out_shape=_STAGE_OUT_SHAPES,
    )(a0, a1, b, wext)


def _tc_stage3(a0, a1, b, wp1, bp1, wp2, bp2):
    return pl.pallas_call(
        _stage3_body,
        grid=(N // BLK,),
        in_specs=[_BLK_SPEC, _BLK_SPEC, _ROW_SPEC,
                  pl.BlockSpec((H, H), lambda i: (0, 0)),
                  _ROW_SPEC,
                  pl.BlockSpec((H, 8), lambda i: (0, 0)),
                  pl.BlockSpec((1, 8), lambda i: (0, 0))],
        out_specs=pl.BlockSpec((BLK, 8), lambda i: (i, 0)),
        out_shape=jax.ShapeDtypeStruct((N, 8), jnp.float32),
    )(a0, a1, b, wp1, bp1, wp2, bp2)


@functools.partial(
    pl.kernel,
    out_type=(jax.ShapeDtypeStruct((EROWS, CHUNK), jnp.float32),
              jax.ShapeDtypeStruct((NP,), jnp.float32)),
    mesh=plsc.VectorSubcoreMesh(core_axis_name="c", subcore_axis_name="s"),
    compiler_params=pltpu.CompilerParams(needs_layout_passes=False),
    scratch_types=[
        pltpu.VMEM((N,), jnp.float32),
        pltpu.VMEM((N,), jnp.float32),
        pltpu.VMEM((RPS, CHUNK), jnp.int32),
        pltpu.VMEM((RPS, CHUNK), jnp.int32),
        pltpu.VMEM((CHUNK,), jnp.float32),
        pltpu.VMEM((8, CHUNK), jnp.float32),
        pltpu.VMEM((NP,), jnp.float32),
        pltpu.VMEM((16,), jnp.float32),
        pltpu.VMEM_SHARED((NP,), jnp.float32),
    ],
)
def _sc_denom_kernel(src_hbm, dst_hbm, ssrc_hbm, sdst_hbm, cvec_hbm,
                     alpha_hbm, den_hbm,
                     ssrc_v, sdst_v, src_v, dst_v, exbuf, astore, denom_v,
                     cvec_v, denom_sh):
    c = lax.axis_index("c")
    s = lax.axis_index("s")
    f32 = jnp.float32

    @pl.when(c == 0)
    def _body():
        pltpu.sync_copy(ssrc_hbm, ssrc_v)
        pltpu.sync_copy(sdst_hbm, sdst_v)
        pltpu.sync_copy(src_hbm.at[pl.ds(s * RPS, RPS)], src_v)
        pltpu.sync_copy(dst_hbm.at[pl.ds(s * RPS, RPS)], dst_v)
        pltpu.sync_copy(cvec_hbm, cvec_v)
        Cv = cvec_v[pl.ds(0, 16)]

        z16 = jnp.zeros((16,), f32)
        for k in range(8):
            exbuf[pl.ds(16 * k, 16)] = z16
        for m in range(5):
            pltpu.sync_copy(
                exbuf, denom_sh.at[pl.ds(640 * s + CHUNK * m, CHUNK)])
        plsc.subcore_barrier()

        iota16 = lax.iota(jnp.int32, 16)

        def _ex_chunk(jl):
            gbase = (s * RPS + jl) * CHUNK
            for k in range(8):
                isrc = src_v[jl, pl.ds(16 * k, 16)]
                idst = dst_v[jl, pl.ds(16 * k, 16)]
                z = (plsc.load_gather(ssrc_v, [isrc])
                     + plsc.load_gather(sdst_v, [idst]))
                a = jnp.where(z > 0, z, 0.2 * z)
                e = jnp.exp(a - Cv)
                e = jnp.where(gbase + 16 * k + iota16 < E, e, 0.0)
                exbuf[pl.ds(16 * k, 16)] = e

        def _den(jl, _):
            _ex_chunk(jl)
            pltpu.sync_copy(exbuf, denom_sh.at[dst_v.at[jl]], add=True)
            return 0
        lax.fori_loop(0, RPS, _den, 0)
        plsc.subcore_barrier()

        pltpu.sync_copy(denom_sh, denom_v)
        pltpu.sync_copy(denom_sh.at[pl.ds(640 * s, 640)],
                        den_hbm.at[pl.ds(640 * s, 640)])

        def _alpha(jo, _):
            for ji in range(8):
                jl = 8 * jo + ji
                _ex_chunk(jl)
                for k in range(8):
                    idst = dst_v[jl, pl.ds(16 * k, 16)]
                    d = plsc.load_gather(denom_v, [idst])
                    astore[ji, pl.ds(16 * k, 16)] = (
                        exbuf[pl.ds(16 * k, 16)] / jnp.maximum(d, 1e-16))
            pltpu.sync_copy(astore, alpha_hbm.at[pl.ds(s * RPS + 8 * jo, 8)])
            return 0
        lax.fori_loop(0, RPS // 8, _alpha, 0)


@functools.partial(
    pl.kernel,
    out_type=jax.ShapeDtypeStruct((2 * NP, H), jnp.float32),
    mesh=plsc.VectorSubcoreMesh(core_axis_name="c", subcore_axis_name="s"),
    compiler_params=pltpu.CompilerParams(needs_layout_passes=False),
    scratch_types=[
        pltpu.VMEM((ROWN, CHUNK), jnp.int32),
        pltpu.VMEM((ROWN, CHUNK), jnp.int32),
        pltpu.VMEM((ROWN, CHUNK), jnp.float32),
        pltpu.VMEM((CHUNK, H), jnp.float32),
        pltpu.VMEM_SHARED((NP, H), jnp.float32),
        pltpu.SemaphoreType.DMA,
    ],
)
def _sc_scatter_kernel(src_hbm, dst_hbm, alpha_hbm, h_hbm, out_hbm,
                       src_v, dst_v, alpha_v, rows_v, acc_sh, sem):
    c = lax.axis_index("c")
    s = lax.axis_index("s")
    f32 = jnp.float32
    base = s * RPS + c * ROWN

    pltpu.sync_copy(src_hbm.at[pl.ds(base, ROWN)], src_v)
    pltpu.sync_copy(dst_hbm.at[pl.ds(base, ROWN)], dst_v)
    pltpu.sync_copy(alpha_hbm.at[pl.ds(base, ROWN)], alpha_v)

    z16 = jnp.zeros((16,), f32)

    def _zr(i, _):
        for k in range(8):
            rows_v[i, pl.ds(16 * k, 16)] = z16
        return 0
    lax.fori_loop(0, CHUNK, _zr, 0)
    for m in range(5):
        pltpu.sync_copy(rows_v, acc_sh.at[pl.ds(640 * s + CHUNK * m, CHUNK)])
    plsc.subcore_barrier()

    def _row(jl, _):
        gather = pltpu.async_copy(h_hbm.at[src_v.at[jl]], rows_v, sem)
        gather.wait()

        def _scale(g, _):
            a16 = alpha_v[jl, pl.ds(16 * g, 16)]
            for l in range(16):
                ab = jnp.full((16,), a16[l], f32)
                i = 16 * g + l
                for k in range(8):
                    rows_v[i, pl.ds(16 * k, 16)] = (
                        rows_v[i, pl.ds(16 * k, 16)] * ab)
            return 0
        lax.fori_loop(0, CHUNK // 16, _scale, 0)
        pltpu.sync_copy(rows_v, acc_sh.at[dst_v.at[jl]], add=True)
        return 0
    lax.fori_loop(0, ROWN, _row, 0)
    plsc.subcore_barrier()

    for m in range(5):
        r = 640 * s + CHUNK * m
        pltpu.sync_copy(acc_sh.at[pl.ds(r, CHUNK)],
                        out_hbm.at[pl.ds(c * NP + r, CHUNK)])


def _edge_phase(src2d, dst2d, s_arr, mx, h):
    cvec = jnp.full((16,), jnp.maximum(mx[0, 0] + mx[0, 1], 0.0), jnp.float32)
    alpha2d, _den = _sc_denom_kernel(src2d, dst2d, s_arr[:, 0], s_arr[:, 1],
                                     cvec)
    return _sc_scatter_kernel(src2d, dst2d, alpha2d, h)


def kernel(discrete_x, continous_x, edge_index, edge_attr, churn_date, t,
           W_c, b_c, W0, b0, W1, a_src1, a_dst1, b1,
           W2, a_src2, a_dst2, b2, Wp1, bp1, Wp2, bp2):
    W0a = W0[:32]
    W0b = W0[32:].reshape(3, 32, H)
    Wfold = jnp.einsum("ij,gjk->gik", W_c, W0b).reshape(96, H)
    Wcat = jnp.concatenate([W0a, Wfold], axis=0)
    bcat = (b0 + jnp.einsum("j,gjk->k", b_c, W0b))[None]

    def wext(W, a_s, a_d):
        cols = jnp.zeros((H, H), jnp.float32)
        cols = cols.at[:, 0].set(W @ a_s).at[:, 1].set(W @ a_d)
        return jnp.concatenate([W, cols], axis=1)

    Wext1 = wext(W1, a_src1, a_dst1)
    Wext2 = wext(W2, a_src2, a_dst2)
    Wp2p = jnp.zeros((H, 8), jnp.float32).at[:, 0].set(Wp2[:, 0])
    bp2p = jnp.zeros((1, 8), jnp.float32).at[0, 0].set(bp2[0])

    X = jnp.concatenate([discrete_x, continous_x.reshape(N, 96)], axis=1)
    pad = jnp.zeros((EPAD - E,), jnp.int32)
    src2d = jnp.concatenate([edge_index[0], pad]).reshape(EROWS, CHUNK)
    dst2d = jnp.concatenate([edge_index[1], pad]).reshape(EROWS, CHUNK)

    h1, s1, mx1 = _tc_stage1(X, Wcat, bcat, Wext1)
    o1 = _edge_phase(src2d, dst2d, s1, mx1, h1)

    h2, s2, mx2 = _tc_stage2(o1[:N], o1[NP:NP + N], b1[None], Wext2)
    o2 = _edge_phase(src2d, dst2d, s2, mx2, h2)

    o3 = _tc_stage3(o2[:N], o2[NP:NP + N], b2[None], Wp1, bp1[None],
                    Wp2p, bp2p)
    return o3[:, :1]


# speedup vs baseline: 9.8579x; 1.0540x over previous
import functools

import jax
import jax.numpy as jnp
from jax import lax
from jax.experimental import pallas as pl
from jax.experimental.pallas import tpu as pltpu
from jax.experimental.pallas import tpu_sc as plsc

N = 10000
E = 320000
H = 128

BLK = 1000

CHUNK = 128
EROWS = 2560
EPAD = EROWS * CHUNK
RPS = EROWS // 16
ROWN = RPS // 2
NP = 10240


def _split_out(out, h_ref, s_ref, mx_ref):
    h_ref[...] = out[:, :H]
    s_blk = out[:, H:]
    s_ref[...] = s_blk
    cur = jnp.max(s_blk, axis=0, keepdims=True)

    @pl.when(pl.program_id(0) == 0)
    def _init():
        mx_ref[...] = cur

    @pl.when(pl.program_id(0) != 0)
    def _acc():
        mx_ref[...] = jnp.maximum(mx_ref[...], cur)


def _stage1_body(x_ref, wcat_ref, bcat_ref, wext_ref, h_ref, s_ref, mx_ref):
    x1 = jnp.maximum(
        jnp.dot(x_ref[...], wcat_ref[...], preferred_element_type=jnp.float32)
        + bcat_ref[...], 0.0)
    out = jnp.dot(x1, wext_ref[...], preferred_element_type=jnp.float32)
    _split_out(out, h_ref, s_ref, mx_ref)


def _stage2_body(a0_ref, a1_ref, b_ref, wext_ref, h_ref, s_ref, mx_ref):
    x = jnp.maximum(a0_ref[...] + a1_ref[...] + b_ref[...], 0.0)
    out = jnp.dot(x, wext_ref[...], preferred_element_type=jnp.float32)
    _split_out(out, h_ref, s_ref, mx_ref)


def _stage3_body(a0_ref, a1_ref, b_ref, wp1_ref, bp1_ref, wp2_ref, bp2_ref,
                 out_ref):
    x = jnp.maximum(a0_ref[...] + a1_ref[...] + b_ref[...], 0.0)
    hm = jnp.maximum(
        jnp.dot(x, wp1_ref[...], preferred_element_type=jnp.float32)
        + bp1_ref[...], 0.0)
    out_ref[...] = (
        jnp.dot(hm, wp2_ref[...], preferred_element_type=jnp.float32)
        + bp2_ref[...])


_BLK_SPEC = pl.BlockSpec((BLK, H), lambda i: (i, 0))
_ROW_SPEC = pl.BlockSpec((1, H), lambda i: (0, 0))
_STAGE_OUT_SPECS = [
    _BLK_SPEC,
    _BLK_SPEC,
    pl.BlockSpec((1, H), lambda i: (0, 0)),
]
_STAGE_OUT_SHAPES = [
    jax.ShapeDtypeStruct((N, H), jnp.float32),
    jax.ShapeDtypeStruct((N, H), jnp.float32),
    jax.ShapeDtypeStruct((1, H), jnp.float32),
]


def _tc_stage1(x, wcat, bcat, wext):
    return pl.pallas_call(
        _stage1_body,
        grid=(N // BLK,),
        in_specs=[
            _BLK_SPEC,
            pl.BlockSpec((H, H), lambda i: (0, 0)),
            _ROW_SPEC,
            pl.BlockSpec((H, 2 * H), lambda i: (0, 0)),
        ],
        out_specs=_STAGE_OUT_SPECS,
        out_shape=_STAGE_OUT_SHAPES,
    )(x, wcat, bcat, wext)


def _tc_stage2(a0, a1, b, wext):
    return pl.pallas_call(
        _stage2_body,
        grid=(N // BLK,),
        in_specs=[_BLK_SPEC, _BLK_SPEC, _ROW_SPEC,
                  pl.BlockSpec((H, 2 * H), lambda i: (0, 0))],
        out_specs=_STAGE_OUT_SPECS,
        out_shape=_STAGE_OUT_SHAPES,
    )(a0, a1, b, wext)


def _tc_stage3(a0, a1, b, wp1, bp1, wp2, bp2):
    return pl.pallas_call(
        _stage3_body,
        grid=(N // BLK,),
        in_specs=[_BLK_SPEC, _BLK_SPEC, _ROW_SPEC,
                  pl.BlockSpec((H, H), lambda i: (0, 0)),
                  _ROW_SPEC,
                  pl.BlockSpec((H, 8), lambda i: (0, 0)),
                  pl.BlockSpec((1, 8), lambda i: (0, 0))],
        out_specs=pl.BlockSpec((BLK, 8), lambda i: (i, 0)),
        out_shape=jax.ShapeDtypeStruct((N, 8), jnp.float32),
    )(a0, a1, b, wp1, bp1, wp2, bp2)


@functools.partial(
    pl.kernel,
    out_type=(jax.ShapeDtypeStruct((EROWS, CHUNK), jnp.float32),
              jax.ShapeDtypeStruct((NP,), jnp.float32)),
    mesh=plsc.VectorSubcoreMesh(core_axis_name="c", subcore_axis_name="s"),
    compiler_params=pltpu.CompilerParams(needs_layout_passes=False),
    scratch_types=[
        pltpu.VMEM((N,), jnp.float32),
        pltpu.VMEM((N,), jnp.float32),
        pltpu.VMEM((RPS, CHUNK), jnp.int32),
        pltpu.VMEM((RPS, CHUNK), jnp.int32),
        pltpu.VMEM((CHUNK,), jnp.float32),
        pltpu.VMEM((8, CHUNK), jnp.float32),
        pltpu.VMEM((NP,), jnp.float32),
        pltpu.VMEM((16,), jnp.float32),
        pltpu.VMEM_SHARED((NP,), jnp.float32),
    ],
)
def _sc_denom_kernel(src_hbm, dst_hbm, ssrc_hbm, sdst_hbm, cvec_hbm,
                     alpha_hbm, den_hbm,
                     ssrc_v, sdst_v, src_v, dst_v, exbuf, astore, denom_v,
                     cvec_v, denom_sh):
    c = lax.axis_index("c")
    s = lax.axis_index("s")
    f32 = jnp.float32

    @pl.when(c == 0)
    def _body():
        pltpu.sync_copy(ssrc_hbm, ssrc_v)
        pltpu.sync_copy(sdst_hbm, sdst_v)
        pltpu.sync_copy(src_hbm.at[pl.ds(s * RPS, RPS)], src_v)
        pltpu.sync_copy(dst_hbm.at[pl.ds(s * RPS, RPS)], dst_v)
        pltpu.sync_copy(cvec_hbm, cvec_v)
        Cv = cvec_v[pl.ds(0, 16)]

        z16 = jnp.zeros((16,), f32)
        for k in range(8):
            exbuf[pl.ds(16 * k, 16)] = z16
        for m in range(5):
            pltpu.sync_copy(
                exbuf, denom_sh.at[pl.ds(640 * s + CHUNK * m, CHUNK)])
        plsc.subcore_barrier()

        iota16 = lax.iota(jnp.int32, 16)

        def _ex_chunk(jl):
            gbase = (s * RPS + jl) * CHUNK
            for k in range(8):
                isrc = src_v[jl, pl.ds(16 * k, 16)]
                idst = dst_v[jl, pl.ds(16 * k, 16)]
                z = (plsc.load_gather(ssrc_v, [isrc])
                     + plsc.load_gather(sdst_v, [idst]))
                a = jnp.where(z > 0, z, 0.2 * z)
                e = jnp.exp(a - Cv)
                e = jnp.where(gbase + 16 * k + iota16 < E, e, 0.0)
                exbuf[pl.ds(16 * k, 16)] = e

        def _den(jl, _):
            _ex_chunk(jl)
            pltpu.sync_copy(exbuf, denom_sh.at[dst_v.at[jl]], add=True)
            return 0
        lax.fori_loop(0, RPS, _den, 0)
        plsc.subcore_barrier()

        pltpu.sync_copy(denom_sh, denom_v)
        pltpu.sync_copy(denom_sh.at[pl.ds(640 * s, 640)],
                        den_hbm.at[pl.ds(640 * s, 640)])

        def _alpha(jo, _):
            for ji in range(8):
                jl = 8 * jo + ji
                _ex_chunk(jl)
                for k in range(8):
                    idst = dst_v[jl, pl.ds(16 * k, 16)]
                    d = plsc.load_gather(denom_v, [idst])
                    astore[ji, pl.ds(16 * k, 16)] = (
                        exbuf[pl.ds(16 * k, 16)] / jnp.maximum(d, 1e-16))
            pltpu.sync_copy(astore, alpha_hbm.at[pl.ds(s * RPS + 8 * jo, 8)])
            return 0
        lax.fori_loop(0, RPS // 8, _alpha, 0)


GRP = 8
NGRP = ROWN // GRP


@functools.partial(
    pl.kernel,
    out_type=jax.ShapeDtypeStruct((2 * NP, H), jnp.float32),
    mesh=plsc.VectorSubcoreMesh(core_axis_name="c", subcore_axis_name="s"),
    compiler_params=pltpu.CompilerParams(needs_layout_passes=False),
    scratch_types=[
        pltpu.VMEM((2, GRP, CHUNK), jnp.int32),
        pltpu.VMEM((2, GRP, CHUNK), jnp.int32),
        pltpu.VMEM((2, GRP, CHUNK), jnp.float32),
        pltpu.VMEM((2 * CHUNK, H), jnp.float32),
        pltpu.VMEM_SHARED((NP, H), jnp.float32),
        pltpu.SemaphoreType.DMA,
        pltpu.SemaphoreType.DMA,
        pltpu.SemaphoreType.DMA,
        pltpu.SemaphoreType.DMA,
        pltpu.SemaphoreType.DMA,
        pltpu.SemaphoreType.DMA,
    ],
)
def _sc_scatter_kernel(src_hbm, dst_hbm, alpha_hbm, h_hbm, out_hbm,
                       srcb, dstb, alb, rows_v, acc_sh,
                       sg0, sg1, ss0, ss1, sp0, sp1):
    c = lax.axis_index("c")
    s = lax.axis_index("s")
    f32 = jnp.float32
    base = s * RPS + c * ROWN
    sg = (sg0, sg1)
    ss = (ss0, ss1)
    sp = (sp0, sp1)

    def rows_slot(ti):
        return rows_v.at[pl.ds(CHUNK * ti, CHUNK)]

    def drain_ss(ti):
        pltpu.make_async_copy(
            h_hbm.at[pl.ds(0, CHUNK)], rows_slot(ti), ss[ti]).wait()

    def drain_sg(ti):
        pltpu.make_async_copy(
            h_hbm.at[pl.ds(0, CHUNK)], rows_slot(ti), sg[ti]).wait()

    def drain_sp(slot):
        pltpu.make_async_copy(
            src_hbm.at[pl.ds(0, GRP)], srcb.at[slot], sp[slot]).wait()
        pltpu.make_async_copy(
            dst_hbm.at[pl.ds(0, GRP)], dstb.at[slot], sp[slot]).wait()
        pltpu.make_async_copy(
            alpha_hbm.at[pl.ds(0, GRP)], alb.at[slot], sp[slot]).wait()

    pltpu.sync_copy(src_hbm.at[pl.ds(base, GRP)], srcb.at[0])
    pltpu.sync_copy(dst_hbm.at[pl.ds(base, GRP)], dstb.at[0])
    pltpu.sync_copy(alpha_hbm.at[pl.ds(base, GRP)], alb.at[0])

    z16 = jnp.zeros((16,), f32)

    def _zr(i, _):
        for k in range(8):
            rows_v[i, pl.ds(16 * k, 16)] = z16
        return 0
    lax.fori_loop(0, CHUNK, _zr, 0)
    for m in range(5):
        pltpu.sync_copy(rows_v.at[pl.ds(0, CHUNK)],
                        acc_sh.at[pl.ds(640 * s + CHUNK * m, CHUNK)])
    plsc.subcore_barrier()

    pltpu.async_copy(h_hbm.at[srcb.at[0, 0]], rows_slot(0), sg[0])

    def _outer(go, _):
        for gslot in range(2):
            g = 2 * go + gslot
            nslot = 1 - gslot
            for ji in range(GRP):
                ti = ji % 2
                roff = CHUNK * ti
                drain_sg(ti)

                def _scale(gg, _):
                    a16 = alb[gslot, ji, pl.ds(16 * gg, 16)]
                    for l in range(16):
                        ab = jnp.full((16,), a16[l], f32)
                        i = roff + 16 * gg + l
                        for k in range(8):
                            rows_v[i, pl.ds(16 * k, 16)] = (
                                rows_v[i, pl.ds(16 * k, 16)] * ab)
                    return 0
                lax.fori_loop(0, CHUNK // 16, _scale, 0)
                pltpu.async_copy(rows_v.at[pl.ds(roff, CHUNK)],
                                 acc_sh.at[dstb.at[gslot, ji]],
                                 ss[ti], add=True)

                if ji == 2:
                    def _prefetch():
                        off = base + GRP * (g + 1)
                        pltpu.async_copy(src_hbm.at[pl.ds(off, GRP)],
                                         srcb.at[nslot], sp[nslot])
                        pltpu.async_copy(dst_hbm.at[pl.ds(off, GRP)],
                                         dstb.at[nslot], sp[nslot])
                        pltpu.async_copy(alpha_hbm.at[pl.ds(off, GRP)],
                                         alb.at[nslot], sp[nslot])
                    if gslot == 0:
                        _prefetch()
                    else:
                        pl.when(go < NGRP // 2 - 1)(_prefetch)

                if ji < GRP - 1:
                    t2 = ti ^ 1
                    if ji == 0:
                        pl.when(g > 0)(lambda: drain_ss(t2))
                    else:
                        drain_ss(t2)
                    pltpu.async_copy(h_hbm.at[srcb.at[gslot, ji + 1]],
                                     rows_slot(t2), sg[t2])
                else:
                    def _fire_next():
                        drain_sp(nslot)
                        drain_ss(0)
                        pltpu.async_copy(h_hbm.at[srcb.at[nslot, 0]],
                                         rows_slot(0), sg[0])
                    if gslot == 0:
                        _fire_next()
                    else:
                        pl.when(go < NGRP // 2 - 1)(_fire_next)
        return 0
    lax.fori_loop(0, NGRP // 2, _outer, 0)

    drain_ss(0)
    drain_ss(1)
    plsc.subcore_barrier()

    for m in range(5):
        r = 640 * s + CHUNK * m
        pltpu.sync_copy(acc_sh.at[pl.ds(r, CHUNK)],
                        out_hbm.at[pl.ds(c * NP + r, CHUNK)])


def _edge_phase(src2d, dst2d, s_arr, mx, h):
    cvec = jnp.full((16,), jnp.maximum(mx[0, 0] + mx[0, 1], 0.0), jnp.float32)
    alpha2d, _den = _sc_denom_kernel(src2d, dst2d, s_arr[:, 0], s_arr[:, 1],
                                     cvec)
    return _sc_scatter_kernel(src2d, dst2d, alpha2d, h)


def kernel(discrete_x, continous_x, edge_index, edge_attr, churn_date, t,
           W_c, b_c, W0, b0, W1, a_src1, a_dst1, b1,
           W2, a_src2, a_dst2, b2, Wp1, bp1, Wp2, bp2):
    W0a = W0[:32]
    W0b = W0[32:].reshape(3, 32, H)
    Wfold = jnp.einsum("ij,gjk->gik", W_c, W0b).reshape(96, H)
    Wcat = jnp.concatenate([W0a, Wfold], axis=0)
    bcat = (b0 + jnp.einsum("j,gjk->k", b_c, W0b))[None]

    def wext(W, a_s, a_d):
        cols = jnp.zeros((H, H), jnp.float32)
        cols = cols.at[:, 0].set(W @ a_s).at[:, 1].set(W @ a_d)
        return jnp.concatenate([W, cols], axis=1)

    Wext1 = wext(W1, a_src1, a_dst1)
    Wext2 = wext(W2, a_src2, a_dst2)
    Wp2p = jnp.zeros((H, 8), jnp.float32).at[:, 0].set(Wp2[:, 0])
    bp2p = jnp.zeros((1, 8), jnp.float32).at[0, 0].set(bp2[0])

    X = jnp.concatenate([discrete_x, continous_x.reshape(N, 96)], axis=1)
    pad = jnp.zeros((EPAD - E,), jnp.int32)
    src2d = jnp.concatenate([edge_index[0], pad]).reshape(EROWS, CHUNK)
    dst2d = jnp.concatenate([edge_index[1], pad]).reshape(EROWS, CHUNK)

    h1, s1, mx1 = _tc_stage1(X, Wcat, bcat, Wext1)
    o1 = _edge_phase(src2d, dst2d, s1, mx1, h1)

    h2, s2, mx2 = _tc_stage2(o1[:N], o1[NP:NP + N], b1[None], Wext2)
    o2 = _edge_phase(src2d, dst2d, s2, mx2, h2)

    o3 = _tc_stage3(o2[:N], o2[NP:NP + N], b2[None], Wp1, bp1[None],
                    Wp2p, bp2p)
    return o3[:, :1]


# speedup vs baseline: 11.2484x; 1.1411x over previous
import functools

import jax
import jax.numpy as jnp
from jax import lax
from jax.experimental import pallas as pl
from jax.experimental.pallas import tpu as pltpu
from jax.experimental.pallas import tpu_sc as plsc

N = 10000
E = 320000
H = 128

BLK = 1000

CHUNK = 128
EROWS = 2560
EPAD = EROWS * CHUNK
RPS = EROWS // 16
ROWN = RPS // 2
NP = 10240


def _split_out(out, h_ref, s_ref, mx_ref):
    h_ref[...] = out[:, :H]
    s_blk = out[:, H:]
    s_ref[...] = s_blk
    cur = jnp.max(s_blk, axis=0, keepdims=True)

    @pl.when(pl.program_id(0) == 0)
    def _init():
        mx_ref[...] = cur

    @pl.when(pl.program_id(0) != 0)
    def _acc():
        mx_ref[...] = jnp.maximum(mx_ref[...], cur)


def _stage1_body(x_ref, wcat_ref, bcat_ref, wext_ref, h_ref, s_ref, mx_ref):
    x1 = jnp.maximum(
        jnp.dot(x_ref[...], wcat_ref[...], preferred_element_type=jnp.float32)
        + bcat_ref[...], 0.0)
    out = jnp.dot(x1, wext_ref[...], preferred_element_type=jnp.float32)
    _split_out(out, h_ref, s_ref, mx_ref)


def _stage2_body(a0_ref, a1_ref, b_ref, wext_ref, h_ref, s_ref, mx_ref):
    x = jnp.maximum(a0_ref[...] + a1_ref[...] + b_ref[...], 0.0)
    out = jnp.dot(x, wext_ref[...], preferred_element_type=jnp.float32)
    _split_out(out, h_ref, s_ref, mx_ref)


def _stage3_body(a0_ref, a1_ref, b_ref, wp1_ref, bp1_ref, wp2_ref, bp2_ref,
                 out_ref):
    x = jnp.maximum(a0_ref[...] + a1_ref[...] + b_ref[...], 0.0)
    hm = jnp.maximum(
        jnp.dot(x, wp1_ref[...], preferred_element_type=jnp.float32)
        + bp1_ref[...], 0.0)
    out_ref[...] = (
        jnp.dot(hm, wp2_ref[...], preferred_element_type=jnp.float32)
        + bp2_ref[...])


_BLK_SPEC = pl.BlockSpec((BLK, H), lambda i: (i, 0))
_ROW_SPEC = pl.BlockSpec((1, H), lambda i: (0, 0))
_STAGE_OUT_SPECS = [
    _BLK_SPEC,
    _BLK_SPEC,
    pl.BlockSpec((1, H), lambda i: (0, 0)),
]
_STAGE_OUT_SHAPES = [
    jax.ShapeDtypeStruct((N, H), jnp.float32),
    jax.ShapeDtypeStruct((N, H), jnp.float32),
    jax.ShapeDtypeStruct((1, H), jnp.float32),
]


def _tc_stage1(x, wcat, bcat, wext):
    return pl.pallas_call(
        _stage1_body,
        grid=(N // BLK,),
        in_specs=[
            _BLK_SPEC,
            pl.BlockSpec((H, H), lambda i: (0, 0)),
            _ROW_SPEC,
            pl.BlockSpec((H, 2 * H), lambda i: (0, 0)),
        ],
        out_specs=_STAGE_OUT_SPECS,
        out_shape=_STAGE_OUT_SHAPES,
    )(x, wcat, bcat, wext)


def _tc_stage2(a0, a1, b, wext):
    return pl.pallas_call(
        _stage2_body,
        grid=(N // BLK,),
        in_specs=[_BLK_SPEC, _BLK_SPEC, _ROW_SPEC,
                  pl.BlockSpec((H, 2 * H), lambda i: (0, 0))],
        out_specs=_STAGE_OUT_SPECS,
        out_shape=_STAGE_OUT_SHAPES,
    )(a0, a1, b, wext)


def _tc_stage3(a0, a1, b, wp1, bp1, wp2, bp2):
    return pl.pallas_call(
        _stage3_body,
        grid=(N // BLK,),
        in_specs=[_BLK_SPEC, _BLK_SPEC, _ROW_SPEC,
                  pl.BlockSpec((H, H), lambda i: (0, 0)),
                  _ROW_SPEC,
                  pl.BlockSpec((H, 8), lambda i: (0, 0)),
                  pl.BlockSpec((1, 8), lambda i: (0, 0))],
        out_specs=pl.BlockSpec((BLK, 8), lambda i: (i, 0)),
        out_shape=jax.ShapeDtypeStruct((N, 8), jnp.float32),
    )(a0, a1, b, wp1, bp1, wp2, bp2)


@functools.partial(
    pl.kernel,
    out_type=(jax.ShapeDtypeStruct((EROWS, CHUNK), jnp.float32),
              jax.ShapeDtypeStruct((NP,), jnp.float32)),
    mesh=plsc.VectorSubcoreMesh(core_axis_name="c", subcore_axis_name="s"),
    compiler_params=pltpu.CompilerParams(needs_layout_passes=False),
    scratch_types=[
        pltpu.VMEM((N,), jnp.float32),
        pltpu.VMEM((N,), jnp.float32),
        pltpu.VMEM((RPS, CHUNK), jnp.int32),
        pltpu.VMEM((RPS, CHUNK), jnp.int32),
        pltpu.VMEM((CHUNK,), jnp.float32),
        pltpu.VMEM((8, CHUNK), jnp.float32),
        pltpu.VMEM((NP,), jnp.float32),
        pltpu.VMEM((16,), jnp.float32),
        pltpu.VMEM_SHARED((NP,), jnp.float32),
    ],
)
def _sc_denom_kernel(src_hbm, dst_hbm, ssrc_hbm, sdst_hbm, cvec_hbm,
                     alpha_hbm, den_hbm,
                     ssrc_v, sdst_v, src_v, dst_v, exbuf, astore, denom_v,
                     cvec_v, denom_sh):
    c = lax.axis_index("c")
    s = lax.axis_index("s")
    f32 = jnp.float32

    @pl.when(c == 0)
    def _body():
        pltpu.sync_copy(ssrc_hbm, ssrc_v)
        pltpu.sync_copy(sdst_hbm, sdst_v)
        pltpu.sync_copy(src_hbm.at[pl.ds(s * RPS, RPS)], src_v)
        pltpu.sync_copy(dst_hbm.at[pl.ds(s * RPS, RPS)], dst_v)
        pltpu.sync_copy(cvec_hbm, cvec_v)
        Cv = cvec_v[pl.ds(0, 16)]

        z16 = jnp.zeros((16,), f32)
        for k in range(8):
            exbuf[pl.ds(16 * k, 16)] = z16
        for m in range(5):
            pltpu.sync_copy(
                exbuf, denom_sh.at[pl.ds(640 * s + CHUNK * m, CHUNK)])
        plsc.subcore_barrier()

        iota16 = lax.iota(jnp.int32, 16)

        def _ex_chunk(jl):
            gbase = (s * RPS + jl) * CHUNK
            for k in range(8):
                isrc = src_v[jl, pl.ds(16 * k, 16)]
                idst = dst_v[jl, pl.ds(16 * k, 16)]
                z = (plsc.load_gather(ssrc_v, [isrc])
                     + plsc.load_gather(sdst_v, [idst]))
                a = jnp.where(z > 0, z, 0.2 * z)
                e = jnp.exp(a - Cv)
                e = jnp.where(gbase + 16 * k + iota16 < E, e, 0.0)
                exbuf[pl.ds(16 * k, 16)] = e

        def _den(jl, _):
            _ex_chunk(jl)
            pltpu.sync_copy(exbuf, denom_sh.at[dst_v.at[jl]], add=True)
            return 0
        lax.fori_loop(0, RPS, _den, 0)
        plsc.subcore_barrier()

        pltpu.sync_copy(denom_sh, denom_v)
        pltpu.sync_copy(denom_sh.at[pl.ds(640 * s, 640)],
                        den_hbm.at[pl.ds(640 * s, 640)])

        def _alpha(jo, _):
            for ji in range(8):
                jl = 8 * jo + ji
                _ex_chunk(jl)
                for k in range(8):
                    idst = dst_v[jl, pl.ds(16 * k, 16)]
                    d = plsc.load_gather(denom_v, [idst])
                    astore[ji, pl.ds(16 * k, 16)] = (
                        exbuf[pl.ds(16 * k, 16)] / jnp.maximum(d, 1e-16))
            pltpu.sync_copy(astore, alpha_hbm.at[pl.ds(s * RPS + 8 * jo, 8)])
            return 0
        lax.fori_loop(0, RPS // 8, _alpha, 0)


HC = 64
ROWN2 = 2 * ROWN
GRP2 = 8
NG2 = ROWN2 // GRP2
NSLOT = 4


@functools.partial(
    pl.kernel,
    out_type=jax.ShapeDtypeStruct((2 * NP, H), jnp.float32),
    mesh=plsc.VectorSubcoreMesh(core_axis_name="c", subcore_axis_name="s"),
    compiler_params=pltpu.CompilerParams(needs_layout_passes=False),
    scratch_types=[
        pltpu.VMEM((2, GRP2, HC), jnp.int32),
        pltpu.VMEM((2, GRP2, HC), jnp.int32),
        pltpu.VMEM((2, GRP2, HC), jnp.float32),
        pltpu.VMEM((NSLOT * HC, H), jnp.float32),
        pltpu.VMEM_SHARED((NP, H), jnp.float32),
        [pltpu.SemaphoreType.DMA] * NSLOT,
        [pltpu.SemaphoreType.DMA] * NSLOT,
        [pltpu.SemaphoreType.DMA] * 2,
    ],
)
def _sc_scatter_kernel(src_hbm, dst_hbm, alpha_hbm, h_hbm, out_hbm,
                       srcb, dstb, alb, rows_v, acc_sh, sg, ss, sp):
    c = lax.axis_index("c")
    s = lax.axis_index("s")
    f32 = jnp.float32
    base = 2 * (s * RPS + c * ROWN)

    def rows_slot(q):
        return rows_v.at[pl.ds(HC * q, HC)]

    def drain(sem, q):
        pltpu.make_async_copy(
            h_hbm.at[pl.ds(0, HC)], rows_slot(q), sem[q]).wait()

    def drain_sp(slot):
        pltpu.make_async_copy(
            src_hbm.at[pl.ds(0, GRP2)], srcb.at[slot], sp[slot]).wait()
        pltpu.make_async_copy(
            dst_hbm.at[pl.ds(0, GRP2)], dstb.at[slot], sp[slot]).wait()
        pltpu.make_async_copy(
            alpha_hbm.at[pl.ds(0, GRP2)], alb.at[slot], sp[slot]).wait()

    pltpu.sync_copy(src_hbm.at[pl.ds(base, GRP2)], srcb.at[0])
    pltpu.sync_copy(dst_hbm.at[pl.ds(base, GRP2)], dstb.at[0])
    pltpu.sync_copy(alpha_hbm.at[pl.ds(base, GRP2)], alb.at[0])

    z16 = jnp.zeros((16,), f32)

    def _zr(i, _):
        for k in range(8):
            rows_v[i, pl.ds(16 * k, 16)] = z16
        return 0
    lax.fori_loop(0, 2 * HC, _zr, 0)
    for m in range(5):
        pltpu.sync_copy(rows_v.at[pl.ds(0, CHUNK)],
                        acc_sh.at[pl.ds(640 * s + CHUNK * m, CHUNK)])
    plsc.subcore_barrier()

    for q in range(NSLOT - 1):
        pltpu.async_copy(h_hbm.at[srcb.at[0, q]], rows_slot(q), sg[q])

    def _outer(go, _):
        for gslot in range(2):
            g = 2 * go + gslot
            nslot = 1 - gslot
            for ji in range(GRP2):
                q = ji % NSLOT
                roff = HC * q
                drain(sg, q)

                def _scale(gg, _):
                    a16 = alb[gslot, ji, pl.ds(16 * gg, 16)]
                    for l in range(16):
                        ab = jnp.full((16,), a16[l], f32)
                        i = roff + 16 * gg + l
                        for k in range(8):
                            rows_v[i, pl.ds(16 * k, 16)] = (
                                rows_v[i, pl.ds(16 * k, 16)] * ab)
                    return 0
                lax.fori_loop(0, HC // 16, _scale, 0)
                pltpu.async_copy(rows_v.at[pl.ds(roff, HC)],
                                 acc_sh.at[dstb.at[gslot, ji]],
                                 ss[q], add=True)

                if ji == 2:
                    def _prefetch():
                        off = base + GRP2 * (g + 1)
                        pltpu.async_copy(src_hbm.at[pl.ds(off, GRP2)],
                                         srcb.at[nslot], sp[nslot])
                        pltpu.async_copy(dst_hbm.at[pl.ds(off, GRP2)],
                                         dstb.at[nslot], sp[nslot])
                        pltpu.async_copy(alpha_hbm.at[pl.ds(off, GRP2)],
                                         alb.at[nslot], sp[nslot])
                    pl.when(g < NG2 - 1)(_prefetch)
                if ji == 5:
                    pl.when(g < NG2 - 1)(lambda: drain_sp(nslot))

                q3 = (ji + 3) % NSLOT
                if ji < GRP2 - 3:
                    idxr = srcb.at[gslot, ji + 3]
                    if ji == 0:
                        pl.when(g > 0)(lambda: drain(ss, q3))
                    else:
                        drain(ss, q3)
                    pltpu.async_copy(h_hbm.at[idxr], rows_slot(q3), sg[q3])
                else:
                    idxr = srcb.at[nslot, ji - 5]

                    def _fire_next():
                        drain(ss, q3)
                        pltpu.async_copy(h_hbm.at[idxr], rows_slot(q3),
                                         sg[q3])
                    pl.when(g < NG2 - 1)(_fire_next)
        return 0
    lax.fori_loop(0, NG2 // 2, _outer, 0)

    for q in range(NSLOT):
        drain(ss, q)
    plsc.subcore_barrier()

    for m in range(5):
        r = 640 * s + CHUNK * m
        pltpu.sync_copy(acc_sh.at[pl.ds(r, CHUNK)],
                        out_hbm.at[pl.ds(c * NP + r, CHUNK)])


def _edge_phase(src2d, dst2d, src64, dst64, s_arr, mx, h):
    cvec = jnp.full((16,), jnp.maximum(mx[0, 0] + mx[0, 1], 0.0), jnp.float32)
    alpha2d, _den = _sc_denom_kernel(src2d, dst2d, s_arr[:, 0], s_arr[:, 1],
                                     cvec)
    return _sc_scatter_kernel(src64, dst64, alpha2d.reshape(2 * EROWS, HC), h)


def kernel(discrete_x, continous_x, edge_index, edge_attr, churn_date, t,
           W_c, b_c, W0, b0, W1, a_src1, a_dst1, b1,
           W2, a_src2, a_dst2, b2, Wp1, bp1, Wp2, bp2):
    W0a = W0[:32]
    W0b = W0[32:].reshape(3, 32, H)
    Wfold = jnp.einsum("ij,gjk->gik", W_c, W0b).reshape(96, H)
    Wcat = jnp.concatenate([W0a, Wfold], axis=0)
    bcat = (b0 + jnp.einsum("j,gjk->k", b_c, W0b))[None]

    def wext(W, a_s, a_d):
        cols = jnp.zeros((H, H), jnp.float32)
        cols = cols.at[:, 0].set(W @ a_s).at[:, 1].set(W @ a_d)
        return jnp.concatenate([W, cols], axis=1)

    Wext1 = wext(W1, a_src1, a_dst1)
    Wext2 = wext(W2, a_src2, a_dst2)
    Wp2p = jnp.zeros((H, 8), jnp.float32).at[:, 0].set(Wp2[:, 0])
    bp2p = jnp.zeros((1, 8), jnp.float32).at[0, 0].set(bp2[0])

    X = jnp.concatenate([discrete_x, continous_x.reshape(N, 96)], axis=1)
    pad = jnp.zeros((EPAD - E,), jnp.int32)
    src2d = jnp.concatenate([edge_index[0], pad]).reshape(EROWS, CHUNK)
    dst2d = jnp.concatenate([edge_index[1], pad]).reshape(EROWS, CHUNK)
    src64 = src2d.reshape(2 * EROWS, HC)
    dst64 = dst2d.reshape(2 * EROWS, HC)

    h1, s1, mx1 = _tc_stage1(X, Wcat, bcat, Wext1)
    o1 = _edge_phase(src2d, dst2d, src64, dst64, s1, mx1, h1)

    h2, s2, mx2 = _tc_stage2(o1[:N], o1[NP:NP + N], b1[None], Wext2)
    o2 = _edge_phase(src2d, dst2d, src64, dst64, s2, mx2, h2)

    o3 = _tc_stage3(o2[:N], o2[NP:NP + N], b2[None], Wp1, bp1[None],
                    Wp2p, bp2p)
    return o3[:, :1]


# speedup vs baseline: 12.0495x; 1.0712x over previous
import functools

import jax
import jax.numpy as jnp
from jax import lax
from jax.experimental import pallas as pl
from jax.experimental.pallas import tpu as pltpu
from jax.experimental.pallas import tpu_sc as plsc

N = 10000
E = 320000
H = 128

BLK = 1000

CHUNK = 128
EROWS = 2560
EPAD = EROWS * CHUNK
RPS = EROWS // 16
ROWN = RPS // 2
NP = 10240


def _split_out(out, h_ref, s_ref, mx_ref):
    h_ref[...] = out[:, :H]
    s_blk = out[:, H:]
    s_ref[...] = s_blk
    cur = jnp.max(s_blk, axis=0, keepdims=True)

    @pl.when(pl.program_id(0) == 0)
    def _init():
        mx_ref[...] = cur

    @pl.when(pl.program_id(0) != 0)
    def _acc():
        mx_ref[...] = jnp.maximum(mx_ref[...], cur)


def _stage1_body(x_ref, wcat_ref, bcat_ref, wext_ref, h_ref, s_ref, mx_ref):
    x1 = jnp.maximum(
        jnp.dot(x_ref[...], wcat_ref[...], preferred_element_type=jnp.float32)
        + bcat_ref[...], 0.0)
    out = jnp.dot(x1, wext_ref[...], preferred_element_type=jnp.float32)
    _split_out(out, h_ref, s_ref, mx_ref)


def _stage2_body(a0_ref, a1_ref, b_ref, wext_ref, h_ref, s_ref, mx_ref):
    x = jnp.maximum(a0_ref[...] + a1_ref[...] + b_ref[...], 0.0)
    out = jnp.dot(x, wext_ref[...], preferred_element_type=jnp.float32)
    _split_out(out, h_ref, s_ref, mx_ref)


def _stage3_body(a0_ref, a1_ref, b_ref, wp1_ref, bp1_ref, wp2_ref, bp2_ref,
                 out_ref):
    x = jnp.maximum(a0_ref[...] + a1_ref[...] + b_ref[...], 0.0)
    hm = jnp.maximum(
        jnp.dot(x, wp1_ref[...], preferred_element_type=jnp.float32)
        + bp1_ref[...], 0.0)
    out_ref[...] = (
        jnp.dot(hm, wp2_ref[...], preferred_element_type=jnp.float32)
        + bp2_ref[...])


_BLK_SPEC = pl.BlockSpec((BLK, H), lambda i: (i, 0))
_ROW_SPEC = pl.BlockSpec((1, H), lambda i: (0, 0))
_STAGE_OUT_SPECS = [
    _BLK_SPEC,
    _BLK_SPEC,
    pl.BlockSpec((1, H), lambda i: (0, 0)),
]
_STAGE_OUT_SHAPES = [
    jax.ShapeDtypeStruct((N, H), jnp.float32),
    jax.ShapeDtypeStruct((N, H), jnp.float32),
    jax.ShapeDtypeStruct((1, H), jnp.float32),
]


def _tc_stage1(x, wcat, bcat, wext):
    return pl.pallas_call(
        _stage1_body,
        grid=(N // BLK,),
        in_specs=[
            _BLK_SPEC,
            pl.BlockSpec((H, H), lambda i: (0, 0)),
            _ROW_SPEC,
            pl.BlockSpec((H, 2 * H), lambda i: (0, 0)),
        ],
        out_specs=_STAGE_OUT_SPECS,
        out_shape=_STAGE_OUT_SHAPES,
    )(x, wcat, bcat, wext)


def _tc_stage2(a0, a1, b, wext):
    return pl.pallas_call(
        _stage2_body,
        grid=(N // BLK,),
        in_specs=[_BLK_SPEC, _BLK_SPEC, _ROW_SPEC,
                  pl.BlockSpec((H, 2 * H), lambda i: (0, 0))],
        out_specs=_STAGE_OUT_SPECS,
        out_shape=_STAGE_OUT_SHAPES,
    )(a0, a1, b, wext)


def _tc_stage3(a0, a1, b, wp1, bp1, wp2, bp2):
    return pl.pallas_call(
        _stage3_body,
        grid=(N // BLK,),
        in_specs=[_BLK_SPEC, _BLK_SPEC, _ROW_SPEC,
                  pl.BlockSpec((H, H), lambda i: (0, 0)),
                  _ROW_SPEC,
                  pl.BlockSpec((H, 8), lambda i: (0, 0)),
                  pl.BlockSpec((1, 8), lambda i: (0, 0))],
        out_specs=pl.BlockSpec((BLK, 8), lambda i: (i, 0)),
        out_shape=jax.ShapeDtypeStruct((N, 8), jnp.float32),
    )(a0, a1, b, wp1, bp1, wp2, bp2)


@functools.partial(
    pl.kernel,
    out_type=(jax.ShapeDtypeStruct((EROWS, CHUNK), jnp.float32),
              jax.ShapeDtypeStruct((NP,), jnp.float32)),
    mesh=plsc.VectorSubcoreMesh(core_axis_name="c", subcore_axis_name="s"),
    compiler_params=pltpu.CompilerParams(needs_layout_passes=False),
    scratch_types=[
        pltpu.VMEM((N,), jnp.float32),
        pltpu.VMEM((N,), jnp.float32),
        pltpu.VMEM((RPS, CHUNK), jnp.int32),
        pltpu.VMEM((RPS, CHUNK), jnp.int32),
        pltpu.VMEM((RPS, CHUNK), jnp.float32),
        pltpu.VMEM((2, 8, CHUNK), jnp.float32),
        pltpu.VMEM((NP,), jnp.float32),
        pltpu.VMEM((16,), jnp.float32),
        pltpu.VMEM_SHARED((NP,), jnp.float32),
        pltpu.SemaphoreType.DMA,
        [pltpu.SemaphoreType.DMA] * 2,
    ],
)
def _sc_denom_kernel(src_hbm, dst_hbm, ssrc_hbm, sdst_hbm, cvec_hbm,
                     alpha_hbm, den_hbm,
                     ssrc_v, sdst_v, src_v, dst_v, ex_v, astore, denom_v,
                     cvec_v, denom_sh, ssd, sfl):
    c = lax.axis_index("c")
    s = lax.axis_index("s")
    f32 = jnp.float32
    LAG = 8

    @pl.when(c == 0)
    def _body():
        pltpu.sync_copy(ssrc_hbm, ssrc_v)
        pltpu.sync_copy(sdst_hbm, sdst_v)
        pltpu.sync_copy(src_hbm.at[pl.ds(s * RPS, RPS)], src_v)
        pltpu.sync_copy(dst_hbm.at[pl.ds(s * RPS, RPS)], dst_v)
        pltpu.sync_copy(cvec_hbm, cvec_v)
        Cv = cvec_v[pl.ds(0, 16)]

        z16 = jnp.zeros((16,), f32)
        for k in range(8):
            ex_v[0, pl.ds(16 * k, 16)] = z16
        for m in range(5):
            pltpu.sync_copy(
                ex_v.at[0], denom_sh.at[pl.ds(640 * s + CHUNK * m, CHUNK)])
        plsc.subcore_barrier()

        iota16 = lax.iota(jnp.int32, 16)

        def drain_ssd():
            pltpu.make_async_copy(
                alpha_hbm.at[pl.ds(0, 1)], ex_v.at[pl.ds(0, 1)], ssd).wait()

        def _den(jl, _):
            gbase = (s * RPS + jl) * CHUNK
            for k in range(8):
                isrc = src_v[jl, pl.ds(16 * k, 16)]
                idst = dst_v[jl, pl.ds(16 * k, 16)]
                z = (plsc.load_gather(ssrc_v, [isrc])
                     + plsc.load_gather(sdst_v, [idst]))
                a = jnp.where(z > 0, z, 0.2 * z)
                e = jnp.exp(a - Cv)
                e = jnp.where(gbase + 16 * k + iota16 < E, e, 0.0)
                ex_v[jl, pl.ds(16 * k, 16)] = e
            pltpu.async_copy(ex_v.at[jl], denom_sh.at[dst_v.at[jl]],
                             ssd, add=True)
            pl.when(jl >= LAG)(drain_ssd)
            return 0
        lax.fori_loop(0, RPS, _den, 0)
        for _ in range(LAG):
            drain_ssd()
        plsc.subcore_barrier()

        pltpu.sync_copy(denom_sh, denom_v)
        pltpu.sync_copy(denom_sh.at[pl.ds(640 * s, 640)],
                        den_hbm.at[pl.ds(640 * s, 640)])

        def _alpha_blk(go, aslot):
            jo = 2 * go + aslot

            @pl.when(go > 0)
            def _():
                pltpu.make_async_copy(
                    alpha_hbm.at[pl.ds(0, 8)], astore.at[aslot],
                    sfl[aslot]).wait()
            for ji in range(8):
                jl = 8 * jo + ji
                for k in range(8):
                    idst = dst_v[jl, pl.ds(16 * k, 16)]
                    d = plsc.load_gather(denom_v, [idst])
                    astore[aslot, ji, pl.ds(16 * k, 16)] = (
                        ex_v[jl, pl.ds(16 * k, 16)] / jnp.maximum(d, 1e-16))
            pltpu.async_copy(astore.at[aslot],
                             alpha_hbm.at[pl.ds(s * RPS + 8 * jo, 8)],
                             sfl[aslot])

        def _alpha(go, _):
            _alpha_blk(go, 0)
            _alpha_blk(go, 1)
            return 0
        lax.fori_loop(0, RPS // 16, _alpha, 0)
        for aslot in range(2):
            pltpu.make_async_copy(
                alpha_hbm.at[pl.ds(0, 8)], astore.at[aslot],
                sfl[aslot]).wait()


HC = 64
ROWN2 = 2 * ROWN
GRP2 = 8
NG2 = ROWN2 // GRP2
NSLOT = 4


@functools.partial(
    pl.kernel,
    out_type=jax.ShapeDtypeStruct((2 * NP, H), jnp.float32),
    mesh=plsc.VectorSubcoreMesh(core_axis_name="c", subcore_axis_name="s"),
    compiler_params=pltpu.CompilerParams(needs_layout_passes=False),
    scratch_types=[
        pltpu.VMEM((2, GRP2, HC), jnp.int32),
        pltpu.VMEM((2, GRP2, HC), jnp.int32),
        pltpu.VMEM((2, GRP2, HC), jnp.float32),
        pltpu.VMEM((NSLOT * HC, H), jnp.float32),
        pltpu.VMEM_SHARED((NP, H), jnp.float32),
        [pltpu.SemaphoreType.DMA] * NSLOT,
        [pltpu.SemaphoreType.DMA] * NSLOT,
        [pltpu.SemaphoreType.DMA] * 2,
    ],
)
def _sc_scatter_kernel(src_hbm, dst_hbm, alpha_hbm, h_hbm, out_hbm,
                       srcb, dstb, alb, rows_v, acc_sh, sg, ss, sp):
    c = lax.axis_index("c")
    s = lax.axis_index("s")
    f32 = jnp.float32
    base = 2 * (s * RPS + c * ROWN)

    def rows_slot(q):
        return rows_v.at[pl.ds(HC * q, HC)]

    def drain(sem, q):
        pltpu.make_async_copy(
            h_hbm.at[pl.ds(0, HC)], rows_slot(q), sem[q]).wait()

    def drain_sp(slot):
        pltpu.make_async_copy(
            src_hbm.at[pl.ds(0, GRP2)], srcb.at[slot], sp[slot]).wait()
        pltpu.make_async_copy(
            dst_hbm.at[pl.ds(0, GRP2)], dstb.at[slot], sp[slot]).wait()
        pltpu.make_async_copy(
            alpha_hbm.at[pl.ds(0, GRP2)], alb.at[slot], sp[slot]).wait()

    pltpu.sync_copy(src_hbm.at[pl.ds(base, GRP2)], srcb.at[0])
    pltpu.sync_copy(dst_hbm.at[pl.ds(base, GRP2)], dstb.at[0])
    pltpu.sync_copy(alpha_hbm.at[pl.ds(base, GRP2)], alb.at[0])

    z16 = jnp.zeros((16,), f32)

    def _zr(i, _):
        for k in range(8):
            rows_v[i, pl.ds(16 * k, 16)] = z16
        return 0
    lax.fori_loop(0, 2 * HC, _zr, 0)
    for m in range(5):
        pltpu.sync_copy(rows_v.at[pl.ds(0, CHUNK)],
                        acc_sh.at[pl.ds(640 * s + CHUNK * m, CHUNK)])
    plsc.subcore_barrier()

    for q in range(NSLOT - 1):
        pltpu.async_copy(h_hbm.at[srcb.at[0, q]], rows_slot(q), sg[q])

    def _outer(go, _):
        for gslot in range(2):
            g = 2 * go + gslot
            nslot = 1 - gslot
            for ji in range(GRP2):
                q = ji % NSLOT
                roff = HC * q
                drain(sg, q)

                def _scale(gg, _):
                    a16 = alb[gslot, ji, pl.ds(16 * gg, 16)]
                    for l in range(16):
                        ab = jnp.full((16,), a16[l], f32)
                        i = roff + 16 * gg + l
                        for k in range(8):
                            rows_v[i, pl.ds(16 * k, 16)] = (
                                rows_v[i, pl.ds(16 * k, 16)] * ab)
                    return 0
                lax.fori_loop(0, HC // 16, _scale, 0)
                pltpu.async_copy(rows_v.at[pl.ds(roff, HC)],
                                 acc_sh.at[dstb.at[gslot, ji]],
                                 ss[q], add=True)

                if ji == 2:
                    def _prefetch():
                        off = base + GRP2 * (g + 1)
                        pltpu.async_copy(src_hbm.at[pl.ds(off, GRP2)],
                                         srcb.at[nslot], sp[nslot])
                        pltpu.async_copy(dst_hbm.at[pl.ds(off, GRP2)],
                                         dstb.at[nslot], sp[nslot])
                        pltpu.async_copy(alpha_hbm.at[pl.ds(off, GRP2)],
                                         alb.at[nslot], sp[nslot])
                    pl.when(g < NG2 - 1)(_prefetch)
                if ji == 5:
                    pl.when(g < NG2 - 1)(lambda: drain_sp(nslot))

                q3 = (ji + 3) % NSLOT
                if ji < GRP2 - 3:
                    idxr = srcb.at[gslot, ji + 3]
                    if ji == 0:
                        pl.when(g > 0)(lambda: drain(ss, q3))
                    else:
                        drain(ss, q3)
                    pltpu.async_copy(h_hbm.at[idxr], rows_slot(q3), sg[q3])
                else:
                    idxr = srcb.at[nslot, ji - 5]

                    def _fire_next():
                        drain(ss, q3)
                        pltpu.async_copy(h_hbm.at[idxr], rows_slot(q3),
                                         sg[q3])
                    pl.when(g < NG2 - 1)(_fire_next)
        return 0
    lax.fori_loop(0, NG2 // 2, _outer, 0)

    for q in range(NSLOT):
        drain(ss, q)
    plsc.subcore_barrier()

    for m in range(5):
        r = 640 * s + CHUNK * m
        pltpu.sync_copy(acc_sh.at[pl.ds(r, CHUNK)],
                        out_hbm.at[pl.ds(c * NP + r, CHUNK)])


def _edge_phase(src2d, dst2d, src64, dst64, s_arr, mx, h):
    cvec = jnp.full((16,), jnp.maximum(mx[0, 0] + mx[0, 1], 0.0), jnp.float32)
    alpha2d, _den = _sc_denom_kernel(src2d, dst2d, s_arr[:, 0], s_arr[:, 1],
                                     cvec)
    return _sc_scatter_kernel(src64, dst64, alpha2d.reshape(2 * EROWS, HC), h)


def kernel(discrete_x, continous_x, edge_index, edge_attr, churn_date, t,
           W_c, b_c, W0, b0, W1, a_src1, a_dst1, b1,
           W2, a_src2, a_dst2, b2, Wp1, bp1, Wp2, bp2):
    W0a = W0[:32]
    W0b = W0[32:].reshape(3, 32, H)
    Wfold = jnp.einsum("ij,gjk->gik", W_c, W0b).reshape(96, H)
    Wcat = jnp.concatenate([W0a, Wfold], axis=0)
    bcat = (b0 + jnp.einsum("j,gjk->k", b_c, W0b))[None]

    def wext(W, a_s, a_d):
        cols = jnp.zeros((H, H), jnp.float32)
        cols = cols.at[:, 0].set(W @ a_s).at[:, 1].set(W @ a_d)
        return jnp.concatenate([W, cols], axis=1)

    Wext1 = wext(W1, a_src1, a_dst1)
    Wext2 = wext(W2, a_src2, a_dst2)
    Wp2p = jnp.zeros((H, 8), jnp.float32).at[:, 0].set(Wp2[:, 0])
    bp2p = jnp.zeros((1, 8), jnp.float32).at[0, 0].set(bp2[0])

    X = jnp.concatenate([discrete_x, continous_x.reshape(N, 96)], axis=1)
    pad = jnp.zeros((EPAD - E,), jnp.int32)
    src2d = jnp.concatenate([edge_index[0], pad]).reshape(EROWS, CHUNK)
    dst2d = jnp.concatenate([edge_index[1], pad]).reshape(EROWS, CHUNK)
    src64 = src2d.reshape(2 * EROWS, HC)
    dst64 = dst2d.reshape(2 * EROWS, HC)

    h1, s1, mx1 = _tc_stage1(X, Wcat, bcat, Wext1)
    o1 = _edge_phase(src2d, dst2d, src64, dst64, s1, mx1, h1)

    h2, s2, mx2 = _tc_stage2(o1[:N], o1[NP:NP + N], b1[None], Wext2)
    o2 = _edge_phase(src2d, dst2d, src64, dst64, s2, mx2, h2)

    o3 = _tc_stage3(o2[:N], o2[NP:NP + N], b2[None], Wp1, bp1[None],
                    Wp2p, bp2p)
    return o3[:, :1]


# speedup vs baseline: 12.5536x; 1.0418x over previous
import functools

import jax
import jax.numpy as jnp
from jax import lax
from jax.experimental import pallas as pl
from jax.experimental.pallas import tpu as pltpu
from jax.experimental.pallas import tpu_sc as plsc

N = 10000
E = 320000
H = 128

BLK = 1000

CHUNK = 128
EROWS = 2560
EPAD = EROWS * CHUNK
RPS = EROWS // 16
ROWN = RPS // 2
NP = 10240


def _split_out(out, h_ref, s_ref, mx_ref):
    h_ref[...] = out[:, :H]
    s_blk = out[:, H:]
    s_ref[...] = s_blk
    cur = jnp.max(s_blk, axis=0, keepdims=True)

    @pl.when(pl.program_id(0) == 0)
    def _init():
        mx_ref[...] = cur

    @pl.when(pl.program_id(0) != 0)
    def _acc():
        mx_ref[...] = jnp.maximum(mx_ref[...], cur)


def _stage1_body(x_ref, wcat_ref, bcat_ref, wext_ref, h_ref, s_ref, mx_ref):
    x1 = jnp.maximum(
        jnp.dot(x_ref[...], wcat_ref[...], preferred_element_type=jnp.float32)
        + bcat_ref[...], 0.0)
    out = jnp.dot(x1, wext_ref[...], preferred_element_type=jnp.float32)
    _split_out(out, h_ref, s_ref, mx_ref)


def _stage2_body(a0_ref, a1_ref, b_ref, wext_ref, h_ref, s_ref, mx_ref):
    x = jnp.maximum(a0_ref[...] + a1_ref[...] + b_ref[...], 0.0)
    out = jnp.dot(x, wext_ref[...], preferred_element_type=jnp.float32)
    _split_out(out, h_ref, s_ref, mx_ref)


def _stage3_body(a0_ref, a1_ref, b_ref, wp1_ref, bp1_ref, wp2_ref, bp2_ref,
                 out_ref):
    x = jnp.maximum(a0_ref[...] + a1_ref[...] + b_ref[...], 0.0)
    hm = jnp.maximum(
        jnp.dot(x, wp1_ref[...], preferred_element_type=jnp.float32)
        + bp1_ref[...], 0.0)
    out_ref[...] = (
        jnp.dot(hm, wp2_ref[...], preferred_element_type=jnp.float32)
        + bp2_ref[...])


_BLK_SPEC = pl.BlockSpec((BLK, H), lambda i: (i, 0))
_ROW_SPEC = pl.BlockSpec((1, H), lambda i: (0, 0))
_STAGE_OUT_SPECS = [
    _BLK_SPEC,
    _BLK_SPEC,
    pl.BlockSpec((1, H), lambda i: (0, 0)),
]
_STAGE_OUT_SHAPES = [
    jax.ShapeDtypeStruct((N, H), jnp.float32),
    jax.ShapeDtypeStruct((N, H), jnp.float32),
    jax.ShapeDtypeStruct((1, H), jnp.float32),
]


def _tc_stage1(x, wcat, bcat, wext):
    return pl.pallas_call(
        _stage1_body,
        grid=(N // BLK,),
        in_specs=[
            _BLK_SPEC,
            pl.BlockSpec((H, H), lambda i: (0, 0)),
            _ROW_SPEC,
            pl.BlockSpec((H, 2 * H), lambda i: (0, 0)),
        ],
        out_specs=_STAGE_OUT_SPECS,
        out_shape=_STAGE_OUT_SHAPES,
    )(x, wcat, bcat, wext)


def _tc_stage2(a0, a1, b, wext):
    return pl.pallas_call(
        _stage2_body,
        grid=(N // BLK,),
        in_specs=[_BLK_SPEC, _BLK_SPEC, _ROW_SPEC,
                  pl.BlockSpec((H, 2 * H), lambda i: (0, 0))],
        out_specs=_STAGE_OUT_SPECS,
        out_shape=_STAGE_OUT_SHAPES,
    )(a0, a1, b, wext)


def _tc_stage3(a0, a1, b, wp1, bp1, wp2, bp2):
    return pl.pallas_call(
        _stage3_body,
        grid=(N // BLK,),
        in_specs=[_BLK_SPEC, _BLK_SPEC, _ROW_SPEC,
                  pl.BlockSpec((H, H), lambda i: (0, 0)),
                  _ROW_SPEC,
                  pl.BlockSpec((H, 8), lambda i: (0, 0)),
                  pl.BlockSpec((1, 8), lambda i: (0, 0))],
        out_specs=pl.BlockSpec((BLK, 8), lambda i: (i, 0)),
        out_shape=jax.ShapeDtypeStruct((N, 8), jnp.float32),
    )(a0, a1, b, wp1, bp1, wp2, bp2)


@functools.partial(
    pl.kernel,
    out_type=(jax.ShapeDtypeStruct((EROWS, CHUNK), jnp.float32),
              jax.ShapeDtypeStruct((NP,), jnp.float32)),
    mesh=plsc.VectorSubcoreMesh(core_axis_name="c", subcore_axis_name="s"),
    compiler_params=pltpu.CompilerParams(needs_layout_passes=False),
    scratch_types=[
        pltpu.VMEM((N,), jnp.float32),
        pltpu.VMEM((N,), jnp.float32),
        pltpu.VMEM((RPS, CHUNK), jnp.int32),
        pltpu.VMEM((RPS, CHUNK), jnp.int32),
        pltpu.VMEM((RPS, CHUNK), jnp.float32),
        pltpu.VMEM((16,), jnp.float32),
        pltpu.VMEM_SHARED((NP,), jnp.float32),
        pltpu.SemaphoreType.DMA,
        pltpu.SemaphoreType.DMA,
    ],
)
def _sc_denom_kernel(src_hbm, dst_hbm, ssrc_hbm, sdst_hbm, cvec_hbm,
                     ex_hbm, den_hbm,
                     ssrc_v, sdst_v, src_v, dst_v, ex_v,
                     cvec_v, denom_sh, ssd, sfl):
    c = lax.axis_index("c")
    s = lax.axis_index("s")
    f32 = jnp.float32
    LAG = 8

    @pl.when(c == 0)
    def _body():
        pltpu.sync_copy(ssrc_hbm, ssrc_v)
        pltpu.sync_copy(sdst_hbm, sdst_v)
        pltpu.sync_copy(src_hbm.at[pl.ds(s * RPS, RPS)], src_v)
        pltpu.sync_copy(dst_hbm.at[pl.ds(s * RPS, RPS)], dst_v)
        pltpu.sync_copy(cvec_hbm, cvec_v)
        Cv = cvec_v[pl.ds(0, 16)]

        z16 = jnp.zeros((16,), f32)
        for k in range(8):
            ex_v[0, pl.ds(16 * k, 16)] = z16
        for m in range(5):
            pltpu.sync_copy(
                ex_v.at[0], denom_sh.at[pl.ds(640 * s + CHUNK * m, CHUNK)])
        plsc.subcore_barrier()

        iota16 = lax.iota(jnp.int32, 16)

        def drain_ssd():
            pltpu.make_async_copy(
                ex_hbm.at[pl.ds(0, 1)], ex_v.at[pl.ds(0, 1)], ssd).wait()

        def _den_blk(go, _):
            for ji in range(8):
                jl = 8 * go + ji
                gbase = (s * RPS + jl) * CHUNK
                for k in range(8):
                    isrc = src_v[jl, pl.ds(16 * k, 16)]
                    idst = dst_v[jl, pl.ds(16 * k, 16)]
                    z = (plsc.load_gather(ssrc_v, [isrc])
                         + plsc.load_gather(sdst_v, [idst]))
                    a = jnp.where(z > 0, z, 0.2 * z)
                    e = jnp.exp(a - Cv)
                    e = jnp.where(gbase + 16 * k + iota16 < E, e, 0.0)
                    ex_v[jl, pl.ds(16 * k, 16)] = e
                pltpu.async_copy(ex_v.at[jl], denom_sh.at[dst_v.at[jl]],
                                 ssd, add=True)
                pl.when(jl >= LAG)(drain_ssd)
            pltpu.async_copy(ex_v.at[pl.ds(8 * go, 8)],
                             ex_hbm.at[pl.ds(s * RPS + 8 * go, 8)], sfl)
            return 0
        lax.fori_loop(0, RPS // 8, _den_blk, 0)
        for _ in range(LAG):
            drain_ssd()
        for _ in range(RPS // 8):
            pltpu.make_async_copy(
                ex_hbm.at[pl.ds(0, 8)], ex_v.at[pl.ds(0, 8)], sfl).wait()
        plsc.subcore_barrier()

        pltpu.sync_copy(denom_sh.at[pl.ds(640 * s, 640)],
                        den_hbm.at[pl.ds(640 * s, 640)])


HC = 64
ROWN2 = 2 * ROWN
GRP2 = 8
NG2 = ROWN2 // GRP2
NSLOT = 4


@functools.partial(
    pl.kernel,
    out_type=jax.ShapeDtypeStruct((2 * NP, H), jnp.float32),
    mesh=plsc.VectorSubcoreMesh(core_axis_name="c", subcore_axis_name="s"),
    compiler_params=pltpu.CompilerParams(needs_layout_passes=False),
    scratch_types=[
        pltpu.VMEM((2, GRP2, HC), jnp.int32),
        pltpu.VMEM((2, GRP2, HC), jnp.int32),
        pltpu.VMEM((2, GRP2, HC), jnp.float32),
        pltpu.VMEM((NSLOT * HC, H), jnp.float32),
        pltpu.VMEM((NP,), jnp.float32),
        pltpu.VMEM_SHARED((NP, H), jnp.float32),
        [pltpu.SemaphoreType.DMA] * NSLOT,
        [pltpu.SemaphoreType.DMA] * NSLOT,
        [pltpu.SemaphoreType.DMA] * 2,
    ],
)
def _sc_scatter_kernel(src_hbm, dst_hbm, alpha_hbm, h_hbm, den_hbm, out_hbm,
                       srcb, dstb, alb, rows_v, denom_v, acc_sh, sg, ss, sp):
    c = lax.axis_index("c")
    s = lax.axis_index("s")
    f32 = jnp.float32
    base = 2 * (s * RPS + c * ROWN)

    def rows_slot(q):
        return rows_v.at[pl.ds(HC * q, HC)]

    def drain(sem, q):
        pltpu.make_async_copy(
            h_hbm.at[pl.ds(0, HC)], rows_slot(q), sem[q]).wait()

    def drain_sp(slot):
        pltpu.make_async_copy(
            src_hbm.at[pl.ds(0, GRP2)], srcb.at[slot], sp[slot]).wait()
        pltpu.make_async_copy(
            dst_hbm.at[pl.ds(0, GRP2)], dstb.at[slot], sp[slot]).wait()
        pltpu.make_async_copy(
            alpha_hbm.at[pl.ds(0, GRP2)], alb.at[slot], sp[slot]).wait()

    pltpu.sync_copy(src_hbm.at[pl.ds(base, GRP2)], srcb.at[0])
    pltpu.sync_copy(dst_hbm.at[pl.ds(base, GRP2)], dstb.at[0])
    pltpu.sync_copy(alpha_hbm.at[pl.ds(base, GRP2)], alb.at[0])
    pltpu.sync_copy(den_hbm, denom_v)

    z16 = jnp.zeros((16,), f32)

    def _zr(i, _):
        for k in range(8):
            rows_v[i, pl.ds(16 * k, 16)] = z16
        return 0
    lax.fori_loop(0, 2 * HC, _zr, 0)
    for m in range(5):
        pltpu.sync_copy(rows_v.at[pl.ds(0, CHUNK)],
                        acc_sh.at[pl.ds(640 * s + CHUNK * m, CHUNK)])
    plsc.subcore_barrier()

    for q in range(NSLOT - 1):
        pltpu.async_copy(h_hbm.at[srcb.at[0, q]], rows_slot(q), sg[q])

    def _outer(go, _):
        for gslot in range(2):
            g = 2 * go + gslot
            nslot = 1 - gslot
            for ji in range(GRP2):
                q = ji % NSLOT
                roff = HC * q
                drain(sg, q)

                def _scale(gg, _):
                    e16 = alb[gslot, ji, pl.ds(16 * gg, 16)]
                    i16 = dstb[gslot, ji, pl.ds(16 * gg, 16)]
                    d16 = plsc.load_gather(denom_v, [i16])
                    a16 = e16 / jnp.maximum(d16, 1e-16)
                    for l in range(16):
                        ab = jnp.full((16,), a16[l], f32)
                        i = roff + 16 * gg + l
                        for k in range(8):
                            rows_v[i, pl.ds(16 * k, 16)] = (
                                rows_v[i, pl.ds(16 * k, 16)] * ab)
                    return 0
                lax.fori_loop(0, HC // 16, _scale, 0)
                pltpu.async_copy(rows_v.at[pl.ds(roff, HC)],
                                 acc_sh.at[dstb.at[gslot, ji]],
                                 ss[q], add=True)

                if ji == 2:
                    def _prefetch():
                        off = base + GRP2 * (g + 1)
                        pltpu.async_copy(src_hbm.at[pl.ds(off, GRP2)],
                                         srcb.at[nslot], sp[nslot])
                        pltpu.async_copy(dst_hbm.at[pl.ds(off, GRP2)],
                                         dstb.at[nslot], sp[nslot])
                        pltpu.async_copy(alpha_hbm.at[pl.ds(off, GRP2)],
                                         alb.at[nslot], sp[nslot])
                    pl.when(g < NG2 - 1)(_prefetch)
                if ji == 5:
                    pl.when(g < NG2 - 1)(lambda: drain_sp(nslot))

                q3 = (ji + 3) % NSLOT
                if ji < GRP2 - 3:
                    idxr = srcb.at[gslot, ji + 3]
                    if ji == 0:
                        pl.when(g > 0)(lambda: drain(ss, q3))
                    else:
                        drain(ss, q3)
                    pltpu.async_copy(h_hbm.at[idxr], rows_slot(q3), sg[q3])
                else:
                    idxr = srcb.at[nslot, ji - 5]

                    def _fire_next():
                        drain(ss, q3)
                        pltpu.async_copy(h_hbm.at[idxr], rows_slot(q3),
                                         sg[q3])
                    pl.when(g < NG2 - 1)(_fire_next)
        return 0
    lax.fori_loop(0, NG2 // 2, _outer, 0)

    for q in range(NSLOT):
        drain(ss, q)
    plsc.subcore_barrier()

    for m in range(5):
        r = 640 * s + CHUNK * m
        pltpu.sync_copy(acc_sh.at[pl.ds(r, CHUNK)],
                        out_hbm.at[pl.ds(c * NP + r, CHUNK)])


def _edge_phase(src2d, dst2d, src64, dst64, s_arr, mx, h):
    cvec = jnp.full((16,), jnp.maximum(mx[0, 0] + mx[0, 1], 0.0), jnp.float32)
    ex2d, den = _sc_denom_kernel(src2d, dst2d, s_arr[:, 0], s_arr[:, 1],
                                 cvec)
    return _sc_scatter_kernel(src64, dst64, ex2d.reshape(2 * EROWS, HC), h,
                              den)


def kernel(discrete_x, continous_x, edge_index, edge_attr, churn_date, t,
           W_c, b_c, W0, b0, W1, a_src1, a_dst1, b1,
           W2, a_src2, a_dst2, b2, Wp1, bp1, Wp2, bp2):
    W0a = W0[:32]
    W0b = W0[32:].reshape(3, 32, H)
    Wfold = jnp.einsum("ij,gjk->gik", W_c, W0b).reshape(96, H)
    Wcat = jnp.concatenate([W0a, Wfold], axis=0)
    bcat = (b0 + jnp.einsum("j,gjk->k", b_c, W0b))[None]

    def wext(W, a_s, a_d):
        cols = jnp.zeros((H, H), jnp.float32)
        cols = cols.at[:, 0].set(W @ a_s).at[:, 1].set(W @ a_d)
        return jnp.concatenate([W, cols], axis=1)

    Wext1 = wext(W1, a_src1, a_dst1)
    Wext2 = wext(W2, a_src2, a_dst2)
    Wp2p = jnp.zeros((H, 8), jnp.float32).at[:, 0].set(Wp2[:, 0])
    bp2p = jnp.zeros((1, 8), jnp.float32).at[0, 0].set(bp2[0])

    X = jnp.concatenate([discrete_x, continous_x.reshape(N, 96)], axis=1)
    pad = jnp.zeros((EPAD - E,), jnp.int32)
    src2d = jnp.concatenate([edge_index[0], pad]).reshape(EROWS, CHUNK)
    dst2d = jnp.concatenate([edge_index[1], pad]).reshape(EROWS, CHUNK)
    src64 = src2d.reshape(2 * EROWS, HC)
    dst64 = dst2d.reshape(2 * EROWS, HC)

    h1, s1, mx1 = _tc_stage1(X, Wcat, bcat, Wext1)
    o1 = _edge_phase(src2d, dst2d, src64, dst64, s1, mx1, h1)

    h2, s2, mx2 = _tc_stage2(o1[:N], o1[NP:NP + N], b1[None], Wext2)
    o2 = _edge_phase(src2d, dst2d, src64, dst64, s2, mx2, h2)

    o3 = _tc_stage3(o2[:N], o2[NP:NP + N], b2[None], Wp1, bp1[None],
                    Wp2p, bp2p)
    return o3[:, :1]
